# Initial kernel scaffold; baseline (speedup 1.0000x reference)
import functools

import jax
import jax.numpy as jnp
from jax import lax
from jax.experimental import pallas as pl
from jax.experimental.pallas import tpu as pltpu
from jax.experimental.pallas import tpu_sc as plsc

NC, NS, L = 2, 16, 16
NW = NC * NS
B, F = 16384, 26
BPW = B // NW
E = BPW * F
NG = BPW // L


def _lr_body(table, idxs, bias16, out, idx_v, vals_v, out_v, bias_v, sem):
    c = lax.axis_index("c")
    s = lax.axis_index("s")
    wid = s * NC + c
    base = wid * E

    pltpu.sync_copy(bias16, bias_v)
    pltpu.sync_copy(idxs.at[pl.ds(base, E)], idx_v)
    pltpu.async_copy(table.at[idx_v], vals_v, sem).wait()

    bias_vec = bias_v[...]

    def group(g, carry):
        addr = (g * L + lax.iota(jnp.int32, (L,))) * F
        acc = bias_vec
        for f in range(F):
            acc = acc + plsc.load_gather(vals_v, [addr + f])
        acc = jnp.clip(acc, -30.0, 30.0)
        out_v[pl.ds(g * L, L)] = 1.0 / (1.0 + jnp.exp(-acc))
        return carry

    lax.fori_loop(0, NG, group, 0)
    pltpu.sync_copy(out_v, out.at[pl.ds(wid * BPW, BPW)])


@jax.jit
def _lr_sc(table, idxs, bias16):
    mesh = plsc.VectorSubcoreMesh(core_axis_name="c", subcore_axis_name="s")
    return pl.kernel(
        _lr_body,
        out_type=jax.ShapeDtypeStruct((B,), jnp.float32),
        mesh=mesh,
        scratch_types=[
            pltpu.VMEM((E,), jnp.int32),
            pltpu.VMEM((E,), jnp.float32),
            pltpu.VMEM((BPW,), jnp.float32),
            pltpu.VMEM((L,), jnp.float32),
            pltpu.SemaphoreType.DMA,
        ],
    )(table, idxs, bias16)


def kernel(inputs, w, bias):
    idxs = inputs.reshape(B * F).astype(jnp.int32)
    table = w.reshape(w.shape[0])
    bias16 = jnp.broadcast_to(bias.astype(jnp.float32), (L,))
    preds = _lr_sc(table, idxs, bias16)
    return preds.reshape(B, 1)


# speedup vs baseline: 1.2838x; 1.2838x over previous
import functools

import jax
import jax.numpy as jnp
from jax import lax
from jax.experimental import pallas as pl
from jax.experimental.pallas import tpu as pltpu
from jax.experimental.pallas import tpu_sc as plsc

NC, NS, L = 2, 16, 16
NW = NC * NS
B, F = 16384, 26
BPW = B // NW
E = BPW * F
NG = BPW // L


def _lr_body(table, idxs, bias16, out, idx_v, vals_v, out_v, bias_v, sem):
    c = lax.axis_index("c")
    s = lax.axis_index("s")
    wid = s * NC + c
    base = wid * E

    pltpu.sync_copy(bias16, bias_v)
    pltpu.sync_copy(idxs.at[pl.ds(base, E)], idx_v)
    pltpu.async_copy(table.at[idx_v], vals_v, sem).wait()

    bias_vec = bias_v[...]

    def group(g, carry):
        addr = (g * L + lax.iota(jnp.int32, L)) * F
        acc = bias_vec
        for f in range(F):
            acc = acc + plsc.load_gather(vals_v, [addr + f])
        acc = jnp.clip(acc, -30.0, 30.0)
        out_v[pl.ds(g * L, L)] = 1.0 / (1.0 + jnp.exp(-acc))
        return carry

    lax.fori_loop(0, NG, group, 0)
    pltpu.sync_copy(out_v, out.at[pl.ds(wid * BPW, BPW)])


@jax.jit
def _lr_sc(table, idxs, bias16):
    mesh = plsc.VectorSubcoreMesh(core_axis_name="c", subcore_axis_name="s")
    return pl.kernel(
        _lr_body,
        out_type=jax.ShapeDtypeStruct((B,), jnp.float32),
        mesh=mesh,
        scratch_types=[
            pltpu.VMEM((E,), jnp.int32),
            pltpu.VMEM((E,), jnp.float32),
            pltpu.VMEM((BPW,), jnp.float32),
            pltpu.VMEM((L,), jnp.float32),
            pltpu.SemaphoreType.DMA,
        ],
        compiler_params=pltpu.CompilerParams(needs_layout_passes=False),
    )(table, idxs, bias16)


def kernel(inputs, w, bias):
    idxs = inputs.reshape(B * F).astype(jnp.int32)
    table = w.reshape(w.shape[0])
    bias16 = jnp.broadcast_to(bias.astype(jnp.float32), (L,))
    preds = _lr_sc(table, idxs, bias16)
    return preds.reshape(B, 1)


# speedup vs baseline: 1.3005x; 1.0130x over previous
import jax
import jax.numpy as jnp
from jax import lax
from jax.experimental import pallas as pl
from jax.experimental.pallas import tpu as pltpu
from jax.experimental.pallas import tpu_sc as plsc

NC, NS, L = 2, 16, 16
NW = NC * NS
B, F = 16384, 26
BPW = B // NW
E = BPW * F
NCH = 4
CH = E // NCH
GPC = CH // F // L


def _lr_body(table, idxs, bias1, out, idx_v, vals_v, out_v, bias_v,
             s0, s1, s2, s3):
    c = lax.axis_index("c")
    s = lax.axis_index("s")
    wid = s * NC + c
    base = wid * E

    pltpu.sync_copy(idxs.at[pl.ds(base, E)], idx_v)
    pltpu.sync_copy(bias1, bias_v.at[pl.ds(0, 1)])

    sems = (s0, s1, s2, s3)
    cps = [
        pltpu.async_copy(
            table.at[idx_v.at[pl.ds(ci * CH, CH)]],
            vals_v.at[pl.ds(ci * CH, CH)],
            sems[ci],
        )
        for ci in range(NCH)
    ]

    b = bias_v[...][0]

    def group(g, carry):
        addr = (g * L + lax.iota(jnp.int32, L)) * F
        acc = jnp.full((L,), b, jnp.float32)
        for f in range(F):
            acc = acc + plsc.load_gather(vals_v, [addr + f])
        acc = jnp.clip(acc, -30.0, 30.0)
        out_v[pl.ds(g * L, L)] = 1.0 / (1.0 + jnp.exp(-acc))
        return carry

    for ci in range(NCH):
        cps[ci].wait()
        lax.fori_loop(ci * GPC, (ci + 1) * GPC, group, 0)

    pltpu.sync_copy(out_v, out.at[pl.ds(wid * BPW, BPW)])


@jax.jit
def _lr_sc(table, idxs, bias1):
    mesh = plsc.VectorSubcoreMesh(core_axis_name="c", subcore_axis_name="s")
    return pl.kernel(
        _lr_body,
        out_type=jax.ShapeDtypeStruct((B,), jnp.float32),
        mesh=mesh,
        scratch_types=[
            pltpu.VMEM((E,), jnp.int32),
            pltpu.VMEM((E,), jnp.float32),
            pltpu.VMEM((BPW,), jnp.float32),
            pltpu.VMEM((L,), jnp.float32),
            pltpu.SemaphoreType.DMA,
            pltpu.SemaphoreType.DMA,
            pltpu.SemaphoreType.DMA,
            pltpu.SemaphoreType.DMA,
        ],
        compiler_params=pltpu.CompilerParams(needs_layout_passes=False),
    )(table, idxs, bias1)


def kernel(inputs, w, bias):
    idxs = inputs.reshape(B * F).astype(jnp.int32)
    table = w.reshape(w.shape[0])
    preds = _lr_sc(table, idxs, bias.astype(jnp.float32))
    return preds.reshape(B, 1)


# speedup vs baseline: 1.4604x; 1.1230x over previous
import jax
import jax.numpy as jnp
from jax import lax
from jax.experimental import pallas as pl
from jax.experimental.pallas import tpu as pltpu
from jax.experimental.pallas import tpu_sc as plsc

NC, NS, L = 2, 16, 16
NW = NC * NS
B, F = 16384, 26
BPW = B // NW
E = BPW * F
NG = BPW // L


def _lr_body(table, idxs, bias1, out, idx_v, vals_v, out_v, bias_v,
             sem_i, sem_g):
    c = lax.axis_index("c")
    s = lax.axis_index("s")
    wid = s * NC + c
    base = wid * BPW

    pltpu.sync_copy(bias1, bias_v.at[pl.ds(0, 1)])
    tbl = table

    idx_cps = [
        pltpu.async_copy(
            idxs.at[pl.ds(f * B + base, BPW)],
            idx_v.at[pl.ds(f * BPW, BPW)],
            sem_i,
        )
        for f in range(F)
    ]
    gat_cps = []
    for f in range(F):
        idx_cps[f].wait()
        gat_cps.append(
            pltpu.async_copy(
                tbl.at[idx_v.at[pl.ds(f * BPW, BPW)]],
                vals_v.at[pl.ds(f * BPW, BPW)],
                sem_g,
            )
        )

    bias_vec = jnp.full((L,), bias_v[...][0], jnp.float32)

    for f in range(F):
        gat_cps[f].wait()
        if f == 0:
            def init_group(g, carry):
                x = vals_v[pl.ds(g * L, L)]
                out_v[pl.ds(g * L, L)] = x + bias_vec
                return carry
            lax.fori_loop(0, NG, init_group, 0)
        else:
            def acc_group(g, carry, f=f):
                x = vals_v[pl.ds(f * BPW + g * L, L)]
                plsc.addupdate(out_v.at[pl.ds(g * L, L)], x)
                return carry
            lax.fori_loop(0, NG, acc_group, 0)

    def sig_group(g, carry):
        x = out_v[pl.ds(g * L, L)]
        x = jnp.clip(x, -30.0, 30.0)
        out_v[pl.ds(g * L, L)] = 1.0 / (1.0 + jnp.exp(-x))
        return carry

    lax.fori_loop(0, NG, sig_group, 0)
    pltpu.sync_copy(out_v, out.at[pl.ds(base, BPW)])


@jax.jit
def _lr_sc(table, idxs, bias1):
    mesh = plsc.VectorSubcoreMesh(core_axis_name="c", subcore_axis_name="s")
    return pl.kernel(
        _lr_body,
        out_type=jax.ShapeDtypeStruct((B,), jnp.float32),
        mesh=mesh,
        scratch_types=[
            pltpu.VMEM((E,), jnp.int32),
            pltpu.VMEM((E,), jnp.float32),
            pltpu.VMEM((BPW,), jnp.float32),
            pltpu.VMEM((L,), jnp.float32),
            pltpu.SemaphoreType.DMA,
            pltpu.SemaphoreType.DMA,
        ],
        compiler_params=pltpu.CompilerParams(needs_layout_passes=False),
    )(table, idxs, bias1)


def kernel(inputs, w, bias):
    idxs = inputs.T.reshape(B * F)
    wlin = w @ jnp.ones((1,), jnp.float32)
    preds = _lr_sc(wlin, idxs.astype(jnp.int32), bias.astype(jnp.float32))
    return preds.reshape(B, 1)


# speedup vs baseline: 1.4675x; 1.0048x over previous
import jax
import jax.numpy as jnp
from jax import lax
from jax.experimental import pallas as pl
from jax.experimental.pallas import tpu as pltpu
from jax.experimental.pallas import tpu_sc as plsc

NC, NS, L = 2, 16, 16
NW = NC * NS
B, F = 16384, 26
BPW = B // NW
E = BPW * F
NG = BPW // L


def _lr_body(table, idxs, bias1, out, idx_v, vals_v, out_v, bias_v,
             sem_i, sem_g):
    c = lax.axis_index("c")
    s = lax.axis_index("s")
    wid = s * NC + c
    base = wid * BPW

    pltpu.sync_copy(bias1, bias_v.at[pl.ds(0, 1)])
    tbl = table

    idx_cps = [
        pltpu.async_copy(
            idxs.at[pl.ds(f * B + base, BPW)],
            idx_v.at[pl.ds(f * BPW, BPW)],
            sem_i,
        )
        for f in range(F)
    ]
    gat_cps = []
    for f in range(F):
        idx_cps[f].wait()
        gat_cps.append(
            pltpu.async_copy(
                tbl.at[idx_v.at[pl.ds(f * BPW, BPW)]],
                vals_v.at[pl.ds(f * BPW, BPW)],
                sem_g,
            )
        )

    bias_vec = jnp.full((L,), bias_v[...][0], jnp.float32)
    iota = lax.iota(jnp.int32, L)
    z16 = jnp.zeros((L,), jnp.int32)

    for f in range(F):
        gat_cps[f].wait()
        if f == 0:
            def init_group(g, carry):
                x = vals_v[pl.ds(g * L, L)]
                out_v[pl.ds(g * L, L)] = x + bias_vec
                return carry
            lax.fori_loop(0, NG, init_group, 0)
        else:
            def acc_group(g, carry, f=f):
                x = vals_v[pl.ds(f * BPW + g * L, L)]
                plsc.addupdate(out_v.at[pl.ds(g * L, L)], x)
                return carry
            lax.fori_loop(0, NG, acc_group, 0)

    def sig_group(g, carry):
        x = out_v[pl.ds(g * L, L)]
        x = jnp.clip(x, -30.0, 30.0)
        out_v[pl.ds(g * L, L)] = 1.0 / (1.0 + jnp.exp(-x))
        return carry

    lax.fori_loop(0, NG, sig_group, 0)
    pltpu.sync_copy(out_v, out.at[pl.ds(base, BPW)])


@jax.jit
def _lr_sc(table, idxs, bias1):
    mesh = plsc.VectorSubcoreMesh(core_axis_name="c", subcore_axis_name="s")
    return pl.kernel(
        _lr_body,
        out_type=jax.ShapeDtypeStruct((B,), jnp.float32),
        mesh=mesh,
        scratch_types=[
            pltpu.VMEM((E,), jnp.int32),
            pltpu.VMEM((E,), jnp.float32),
            pltpu.VMEM((BPW,), jnp.float32),
            pltpu.VMEM((L,), jnp.float32),
            pltpu.SemaphoreType.DMA,
            pltpu.SemaphoreType.DMA,
        ],
        compiler_params=pltpu.CompilerParams(
            needs_layout_passes=False, use_tc_tiling_on_sc=False
        ),
    )(table, idxs, bias1)


def kernel(inputs, w, bias):
    idxs = inputs.T.reshape(B * F)
    wlin = w @ jnp.ones((1,), jnp.float32)
    preds = _lr_sc(wlin, idxs.astype(jnp.int32), bias.astype(jnp.float32))
    return preds.reshape(B, 1)
